# Initial kernel scaffold; baseline (speedup 1.0000x reference)
import jax
import jax.numpy as jnp
from jax.experimental import pallas as pl


def kernel(x, Wr, Wk, Wc, bc):
    raise NotImplementedError("write your pallas kernel here")


# speedup vs baseline: 1.6527x; 1.6527x over previous
import functools

import jax
import jax.numpy as jnp
from jax.experimental import pallas as pl

C = 96
K = 3
TOPK = C * K


def _dw(x, W):
    return jax.lax.conv_general_dilated(
        x, W, window_strides=(1, 1), padding=((1, 1), (1, 1)),
        dimension_numbers=("NCHW", "OIHW", "NCHW"), feature_group_count=C)


def _mm_kernel(s_ref, w_ref, b_ref, o_ref):
    o_ref[...] = jax.lax.dot_general(
        w_ref[...], s_ref[...], (((1,), (0,)), ((), ())),
        preferred_element_type=jnp.float32) + b_ref[...]


def kernel(x, Wr, Wk, Wc, bc):
    N, _, H, Wsp = x.shape
    HW = H * Wsp
    zr = _dw(x, Wr).reshape(C * K * K, HW)
    zk = _dw(x, Wk).reshape(C * K * K, HW)
    r = jax.nn.softmax(zr, axis=0)
    rt = r.T
    tv, ti = jax.lax.top_k(rt, TOPK)
    tk = jnp.take_along_axis(zk.T, ti, axis=-1)
    s = jnp.concatenate([tv, tk], axis=-1).T

    Wm = Wc[:, :, 0, 0]
    bcol = bc[:, None]
    PB = 3584
    y = pl.pallas_call(
        _mm_kernel,
        grid=(HW // PB,),
        in_specs=[
            pl.BlockSpec((2 * TOPK, PB), lambda i: (0, i)),
            pl.BlockSpec((C, 2 * TOPK), lambda i: (0, 0)),
            pl.BlockSpec((C, 1), lambda i: (0, 0)),
        ],
        out_specs=pl.BlockSpec((C, PB), lambda i: (0, i)),
        out_shape=jax.ShapeDtypeStruct((C, HW), jnp.float32),
    )(s, Wm, bcol)
    return y.reshape(N, C, H, Wsp)
